# baseline (speedup 1.0000x reference)
import functools
import jax
import jax.numpy as jnp
from jax.experimental import pallas as pl
from jax.experimental.pallas import tpu as pltpu
from jax.experimental.pallas import tpu_sc as plsc

_N = 32
_K = 12
_W = 2 ** _K
_B = 512
_NW = 32
_RPW = _B // _NW


def _splat_lane(v, lane):
    idx = jnp.full((16, 1), lane, dtype=jnp.int32)
    return jax.lax.gather(
        v, idx,
        jax.lax.GatherDimensionNumbers(
            offset_dims=(), collapsed_slice_dims=(0,), start_index_map=(0,)),
        (1,), mode=jax.lax.GatherScatterMode.PROMISE_IN_BOUNDS)


def _sc_body(x_hbm, g_hbm, out_hbm, x_v, g_v, sgn_v, idx_v, f_v, ob_v, sem):
    wid = jax.lax.axis_index("s") * 2 + jax.lax.axis_index("c")
    base = wid * (_RPW * _N)

    pltpu.sync_copy(x_hbm.at[pl.ds(base, _RPW * _N)], x_v)
    pltpu.sync_copy(g_hbm, g_v)

    lanes = jax.lax.iota(jnp.int32, 16)
    for h in range(2):
        m = jnp.zeros((16,), jnp.int32)
        for j in range(_K):
            m = m + (g_v[pl.ds(j * _N + h * 16, 16)] << j)
        for k in range(16):
            n = h * 16 + k
            msk = _splat_lane(m, k)
            v = lanes & msk & 15
            v = v ^ (v >> 2)
            v = v ^ (v >> 1)
            sgn = (1 - 2 * (v & 1)).astype(jnp.float32)
            tgt = ((msk >> 4) << 4) + lanes
            sgn_v[pl.ds(n * 16, 16)] = sgn
            idx_v[pl.ds(n * 16, 16)] = tgt

    def row_body(r, _):
        def zero_body(i, _c):
            f_v[pl.ds(i * 16, 16)] = jnp.zeros((16,), jnp.float32)
            return _c
        jax.lax.fori_loop(0, _W // 16, zero_body, 0)

        for half in range(2):
            xh = x_v[pl.ds(r * _N + half * 16, 16)]
            for k in range(16):
                n = half * 16 + k
                val = _splat_lane(xh, k) * sgn_v[pl.ds(n * 16, 16)]
                plsc.addupdate_scatter(f_v, [idx_v[pl.ds(n * 16, 16)]], val)

        for s in range(8):
            span = 1 << s

            def fwht_body(t, _c, s=s, span=span):
                p = (((t >> s) << (s + 1)) | (t & (span - 1))) * 16
                q = p + span * 16
                a = f_v[pl.ds(p, 16)]
                b = f_v[pl.ds(q, 16)]
                f_v[pl.ds(p, 16)] = a + b
                f_v[pl.ds(q, 16)] = a - b
                return _c
            jax.lax.fori_loop(0, 128, fwht_body, 0)

        def amax_body(i, carry):
            bv, bb = carry
            v = f_v[pl.ds(i * 16, 16)]
            m = v > bv
            bv = jnp.where(m, v, bv)
            bb = jnp.where(m, jnp.full((16,), 1, jnp.int32) * i, bb)
            return bv, bb
        bv0 = jnp.full((16,), -3.0e38, jnp.float32)
        bb0 = jnp.zeros((16,), jnp.int32)
        bv, bb = jax.lax.fori_loop(0, _W // 16, amax_body, (bv0, bb0))

        maxv = jnp.max(bv)
        wcand = bb * 16 + lanes
        wsel = jnp.where(bv == maxv, wcand, jnp.full((16,), _W, jnp.int32))
        winner = jnp.min(wsel)

        ob = ob_v[...]
        ob_v[...] = jnp.where(lanes == r, jnp.full((16,), 1, jnp.int32) * winner, ob)
        return 0

    ob_v[...] = jnp.zeros((16,), jnp.int32)
    jax.lax.fori_loop(0, _RPW, row_body, 0)

    pltpu.sync_copy(ob_v, out_hbm.at[pl.ds(wid * _RPW, _RPW)])


def kernel(noisy_symbols, G, sigma2):
    x = (noisy_symbols.astype(jnp.float32) * (-4.0 / sigma2[0])).reshape(-1)
    gflat = G.astype(jnp.int32).reshape(-1)

    mesh = plsc.VectorSubcoreMesh(core_axis_name="c", subcore_axis_name="s")
    idx = pl.kernel(
        _sc_body,
        mesh=mesh,
        compiler_params=pltpu.CompilerParams(needs_layout_passes=False),
        out_type=jax.ShapeDtypeStruct((_B,), jnp.int32),
        scratch_types=[
            pltpu.VMEM((_RPW * _N,), jnp.float32),
            pltpu.VMEM((_K * _N,), jnp.int32),
            pltpu.VMEM((_N * 16,), jnp.float32),
            pltpu.VMEM((_N * 16,), jnp.int32),
            pltpu.VMEM((_W,), jnp.float32),
            pltpu.VMEM((16,), jnp.int32),
            pltpu.SemaphoreType.DMA,
        ],
    )(x, gflat)

    jbit = jnp.arange(_K, dtype=jnp.int32)[None, :]
    return ((idx[:, None] >> jbit) & 1).astype(jnp.float32)


# speedup vs baseline: 3.1379x; 3.1379x over previous
import functools
import jax
import jax.numpy as jnp
from jax.experimental import pallas as pl
from jax.experimental.pallas import tpu as pltpu
from jax.experimental.pallas import tpu_sc as plsc

_N = 32
_K = 12
_W = 2 ** _K
_B = 512
_NW = 32
_RPW = _B // _NW


def _splat_lane(v, lane):
    idx = jnp.full((16, 1), lane, dtype=jnp.int32)
    return jax.lax.gather(
        v, idx,
        jax.lax.GatherDimensionNumbers(
            offset_dims=(), collapsed_slice_dims=(0,), start_index_map=(0,)),
        (1,), mode=jax.lax.GatherScatterMode.PROMISE_IN_BOUNDS)


def _sc_body(x_hbm, g_hbm, out_hbm, x_v, g_v, sgn_v, idx_v, f_v, ob_v, sem):
    wid = jax.lax.axis_index("s") * 2 + jax.lax.axis_index("c")
    base = wid * (_RPW * _N)

    pltpu.sync_copy(x_hbm.at[pl.ds(base, _RPW * _N)], x_v)
    pltpu.sync_copy(g_hbm, g_v)

    lanes = jax.lax.iota(jnp.int32, 16)
    for h in range(2):
        m = jnp.zeros((16,), jnp.int32)
        for j in range(_K):
            m = m + (g_v[pl.ds(j * _N + h * 16, 16)] << j)
        for k in range(16):
            n = h * 16 + k
            msk = _splat_lane(m, k)
            v = lanes & msk & 15
            v = v ^ (v >> 2)
            v = v ^ (v >> 1)
            sgn = (1 - 2 * (v & 1)).astype(jnp.float32)
            tgt = ((msk >> 4) << 4) + lanes
            sgn_v[pl.ds(n * 16, 16)] = sgn
            idx_v[pl.ds(n * 16, 16)] = tgt

    def row_body(r, _):
        def zero_body(i, _c):
            for k in range(8):
                f_v[pl.ds((i * 8 + k) * 16, 16)] = jnp.zeros((16,), jnp.float32)
            return _c
        jax.lax.fori_loop(0, _W // 128, zero_body, 0)

        for half in range(2):
            xh = x_v[pl.ds(r * _N + half * 16, 16)]
            for k in range(16):
                n = half * 16 + k
                val = _splat_lane(xh, k) * sgn_v[pl.ds(n * 16, 16)]
                plsc.addupdate_scatter(f_v, [idx_v[pl.ds(n * 16, 16)]], val)

        for s in range(8):
            span = 1 << s

            def fwht_body(t8, _c, s=s, span=span):
                for k in range(8):
                    t = t8 * 8 + k
                    p = (((t >> s) << (s + 1)) | (t & (span - 1))) * 16
                    q = p + span * 16
                    a = f_v[pl.ds(p, 16)]
                    b = f_v[pl.ds(q, 16)]
                    f_v[pl.ds(p, 16)] = a + b
                    f_v[pl.ds(q, 16)] = a - b
                return _c
            jax.lax.fori_loop(0, 16, fwht_body, 0)

        def amax_body(i8, carry):
            bv, bb = carry
            for k in range(8):
                i = i8 * 8 + k
                v = f_v[pl.ds(i * 16, 16)]
                m = v > bv
                bv = jnp.where(m, v, bv)
                bb = jnp.where(m, jnp.full((16,), 1, jnp.int32) * i, bb)
            return bv, bb
        bv0 = jnp.full((16,), -3.0e38, jnp.float32)
        bb0 = jnp.zeros((16,), jnp.int32)
        bv, bb = jax.lax.fori_loop(0, _W // 128, amax_body, (bv0, bb0))

        maxv = jnp.max(bv)
        wcand = bb * 16 + lanes
        wsel = jnp.where(bv == maxv, wcand, jnp.full((16,), _W, jnp.int32))
        winner = jnp.min(wsel)

        ob = ob_v[...]
        ob_v[...] = jnp.where(lanes == r, jnp.full((16,), 1, jnp.int32) * winner, ob)
        return 0

    ob_v[...] = jnp.zeros((16,), jnp.int32)
    jax.lax.fori_loop(0, _RPW, row_body, 0)

    pltpu.sync_copy(ob_v, out_hbm.at[pl.ds(wid * _RPW, _RPW)])


def kernel(noisy_symbols, G, sigma2):
    x = (noisy_symbols.astype(jnp.float32) * (-4.0 / sigma2[0])).reshape(-1)
    gflat = G.astype(jnp.int32).reshape(-1)

    mesh = plsc.VectorSubcoreMesh(core_axis_name="c", subcore_axis_name="s")
    idx = pl.kernel(
        _sc_body,
        mesh=mesh,
        compiler_params=pltpu.CompilerParams(needs_layout_passes=False),
        out_type=jax.ShapeDtypeStruct((_B,), jnp.int32),
        scratch_types=[
            pltpu.VMEM((_RPW * _N,), jnp.float32),
            pltpu.VMEM((_K * _N,), jnp.int32),
            pltpu.VMEM((_N * 16,), jnp.float32),
            pltpu.VMEM((_N * 16,), jnp.int32),
            pltpu.VMEM((_W,), jnp.float32),
            pltpu.VMEM((16,), jnp.int32),
            pltpu.SemaphoreType.DMA,
        ],
    )(x, gflat)

    jbit = jnp.arange(_K, dtype=jnp.int32)[None, :]
    return ((idx[:, None] >> jbit) & 1).astype(jnp.float32)


# speedup vs baseline: 6.4367x; 2.0513x over previous
import jax
import jax.numpy as jnp
from jax.experimental import pallas as pl
from jax.experimental.pallas import tpu as pltpu
from jax.experimental.pallas import tpu_sc as plsc

_N = 32
_K = 12
_W = 2 ** _K
_B = 512
_NW = 32
_RPW = 1
_B_SC = _NW * _RPW
_B_TC = _B - _B_SC


def _splat_lane(v, lane):
    idx = jnp.full((16, 1), lane, dtype=jnp.int32)
    return jax.lax.gather(
        v, idx,
        jax.lax.GatherDimensionNumbers(
            offset_dims=(), collapsed_slice_dims=(0,), start_index_map=(0,)),
        (1,), mode=jax.lax.GatherScatterMode.PROMISE_IN_BOUNDS)


def _sc_body(x_hbm, g_hbm, out_hbm, x_v, g_v, sgn_v, idx_v, f_v, ob_v):
    wid = jax.lax.axis_index("s") * 2 + jax.lax.axis_index("c")
    base = wid * (_RPW * _N)

    pltpu.sync_copy(x_hbm.at[pl.ds(base, _RPW * _N)], x_v)
    pltpu.sync_copy(g_hbm, g_v)

    lanes = jax.lax.iota(jnp.int32, 16)
    for h in range(2):
        m = jnp.zeros((16,), jnp.int32)
        for j in range(_K):
            m = m + (g_v[pl.ds(j * _N + h * 16, 16)] << j)
        for k in range(16):
            n = h * 16 + k
            msk = _splat_lane(m, k)
            v = lanes & msk & 15
            v = v ^ (v >> 2)
            v = v ^ (v >> 1)
            sgn = (1 - 2 * (v & 1)).astype(jnp.float32)
            tgt = ((msk >> 4) << 4) + lanes
            sgn_v[pl.ds(n * 16, 16)] = sgn
            idx_v[pl.ds(n * 16, 16)] = tgt

    def row_body(r, _):
        def zero_body(i, _c):
            for k in range(8):
                f_v[pl.ds((i * 8 + k) * 16, 16)] = jnp.zeros((16,), jnp.float32)
            return _c
        jax.lax.fori_loop(0, _W // 128, zero_body, 0)

        for half in range(2):
            xh = x_v[pl.ds(r * _N + half * 16, 16)]
            for k in range(16):
                n = half * 16 + k
                val = _splat_lane(xh, k) * sgn_v[pl.ds(n * 16, 16)]
                plsc.addupdate_scatter(f_v, [idx_v[pl.ds(n * 16, 16)]], val)

        for s in range(8):
            span = 1 << s

            def fwht_body(t8, _c, s=s, span=span):
                for k in range(8):
                    t = t8 * 8 + k
                    p = (((t >> s) << (s + 1)) | (t & (span - 1))) * 16
                    q = p + span * 16
                    a = f_v[pl.ds(p, 16)]
                    b = f_v[pl.ds(q, 16)]
                    f_v[pl.ds(p, 16)] = a + b
                    f_v[pl.ds(q, 16)] = a - b
                return _c
            jax.lax.fori_loop(0, 16, fwht_body, 0)

        def amax_body(i8, carry):
            bv, bb = carry
            for k in range(8):
                i = i8 * 8 + k
                v = f_v[pl.ds(i * 16, 16)]
                m = v > bv
                bv = jnp.where(m, v, bv)
                bb = jnp.where(m, jnp.full((16,), 1, jnp.int32) * i, bb)
            return bv, bb
        bv0 = jnp.full((16,), -3.0e38, jnp.float32)
        bb0 = jnp.zeros((16,), jnp.int32)
        bv, bb = jax.lax.fori_loop(0, _W // 128, amax_body, (bv0, bb0))

        maxv = jnp.max(bv)
        wcand = bb * 16 + lanes
        wsel = jnp.where(bv == maxv, wcand, jnp.full((16,), _W, jnp.int32))
        winner = jnp.min(wsel)

        ob = ob_v[...]
        ob_v[...] = jnp.where(lanes == r, jnp.full((16,), 1, jnp.int32) * winner, ob)
        return 0

    ob_v[...] = jnp.zeros((16,), jnp.int32)
    jax.lax.fori_loop(0, _RPW, row_body, 0)

    pltpu.sync_copy(ob_v, out_hbm.at[pl.ds(wid * 16, 16)])


def _tc_body(x_ref, g_ref, idx_ref):
    gf = g_ref[...].astype(jnp.float32)
    w_ids = jax.lax.broadcasted_iota(jnp.int32, (_K, _W), 1)
    j_ids = jax.lax.broadcasted_iota(jnp.int32, (_K, _W), 0)
    bits_t = ((w_ids >> j_ids) & 1).astype(jnp.float32)
    c_t = jax.lax.dot_general(
        gf, bits_t, (((0,), (0,)), ((), ())),
        preferred_element_type=jnp.float32)
    c_t = c_t - 2.0 * jnp.floor(c_t * 0.5)
    s_bf = (1.0 - 2.0 * c_t).astype(jnp.bfloat16)
    sc = jnp.concatenate([s_bf, s_bf, s_bf], axis=0)

    x = x_ref[...]
    x1 = x.astype(jnp.bfloat16)
    r1 = x - x1.astype(jnp.float32)
    x2 = r1.astype(jnp.bfloat16)
    x3 = (r1 - x2.astype(jnp.float32)).astype(jnp.bfloat16)
    xc = jnp.concatenate([x1, x2, x3], axis=1)
    scores = jnp.dot(xc, sc, preferred_element_type=jnp.float32)

    idx_ref[...] = jnp.argmax(scores, axis=1).astype(jnp.int32)[:, None]


def kernel(noisy_symbols, G, sigma2):
    x = noisy_symbols.astype(jnp.float32) * (-4.0 / sigma2[0])
    x_sc = x[_B_TC:].reshape(-1)
    gflat = G.astype(jnp.int32).reshape(-1)

    mesh = plsc.VectorSubcoreMesh(core_axis_name="c", subcore_axis_name="s")
    idx_sc_raw = pl.kernel(
        _sc_body,
        mesh=mesh,
        compiler_params=pltpu.CompilerParams(needs_layout_passes=False),
        out_type=jax.ShapeDtypeStruct((_NW * 16,), jnp.int32),
        scratch_types=[
            pltpu.VMEM((_RPW * _N,), jnp.float32),
            pltpu.VMEM((_K * _N,), jnp.int32),
            pltpu.VMEM((_N * 16,), jnp.float32),
            pltpu.VMEM((_N * 16,), jnp.int32),
            pltpu.VMEM((_W,), jnp.float32),
            pltpu.VMEM((16,), jnp.int32),
        ],
    )(x_sc, gflat)

    idx_tc = pl.pallas_call(
        _tc_body,
        out_shape=jax.ShapeDtypeStruct((_B_TC, 1), jnp.int32),
    )(x[:_B_TC], G)

    idx_sc = idx_sc_raw.reshape(_NW, 16)[:, :_RPW].reshape(-1)
    idx = jnp.concatenate([idx_tc[:, 0], idx_sc])

    jbit = jnp.arange(_K, dtype=jnp.int32)[None, :]
    return ((idx[:, None] >> jbit) & 1).astype(jnp.float32)


# speedup vs baseline: 21.8796x; 3.3992x over previous
import jax
import jax.numpy as jnp
from jax.experimental import pallas as pl

_N = 32
_K = 12
_W = 2 ** _K


def _decode_kernel(noisy_ref, g_ref, sig_ref, out_ref):
    gf = g_ref[...].astype(jnp.float32)
    w_ids = jax.lax.broadcasted_iota(jnp.int32, (_K, _W), 1)
    j_ids = jax.lax.broadcasted_iota(jnp.int32, (_K, _W), 0)
    bits_t = ((w_ids >> j_ids) & 1).astype(jnp.float32)
    c_t = jax.lax.dot_general(
        gf, bits_t, (((0,), (0,)), ((), ())),
        preferred_element_type=jnp.float32)
    c_t = c_t - 2.0 * jnp.floor(c_t * 0.5)
    s_bf = (1.0 - 2.0 * c_t).astype(jnp.bfloat16)
    sc = jnp.concatenate([s_bf, s_bf, s_bf], axis=0)

    x = noisy_ref[...] * (-4.0 / sig_ref[0, 0])
    x1 = x.astype(jnp.bfloat16)
    r1 = x - x1.astype(jnp.float32)
    x2 = r1.astype(jnp.bfloat16)
    x3 = (r1 - x2.astype(jnp.float32)).astype(jnp.bfloat16)
    xc = jnp.concatenate([x1, x2, x3], axis=1)
    scores = jnp.dot(xc, sc, preferred_element_type=jnp.float32)

    idx = jnp.argmax(scores, axis=1).astype(jnp.int32)[:, None]

    jbit = jax.lax.broadcasted_iota(jnp.int32, (scores.shape[0], _K), 1)
    out_ref[...] = ((idx >> jbit) & 1).astype(jnp.float32)


def kernel(noisy_symbols, G, sigma2):
    b = noisy_symbols.shape[0]
    sig = jnp.reshape(sigma2.astype(jnp.float32), (1, 1))
    return pl.pallas_call(
        _decode_kernel,
        out_shape=jax.ShapeDtypeStruct((b, _K), jnp.float32),
    )(noisy_symbols, G, sig)
